# Initial kernel scaffold; baseline (speedup 1.0000x reference)
import jax
import jax.numpy as jnp
from jax.experimental import pallas as pl


def kernel(user_table, item_table, u_w, i_w, u_cat_w, i_cat_w, edge_val, edge_user, edge_item):
    raise NotImplementedError("write your pallas kernel here")


# speedup vs baseline: 2.1694x; 2.1694x over previous
import functools

import jax
import jax.numpy as jnp
from jax import lax
from jax.experimental import pallas as pl
from jax.experimental.pallas import tpu as pltpu
from jax.experimental.pallas import tpu_sc as plsc

U = 100000
I = 50000
H = 64
NB = 3
E = 1000000
NL = 2

NC = 2
NS = 16
LANES = 16

E_PAD = 1048576
EPT = E_PAD // NS
SCAN = 2048
NSCAN = EPT // SCAN
GROUPS = SCAN // LANES

CHUNK = 25600
U_PAD = 4 * CHUNK
I_PAD = 2 * CHUNK
ROWS_PT = CHUNK // NS
ZROWS = 64

BATCH = 64
CCAP = SCAN + BATCH


def _sigmoid(x):
    return 1.0 / (1.0 + jnp.exp(-x))


def _iota16():
    return lax.broadcasted_iota(jnp.int32, (LANES,), 0)


def _emit_pass(dest_hbm, src_hbm, val_hbm, table_hbm, out_hbm, base, ebase,
               tile,
               dest_v, srcv_v, val_v, cidx, cdst, cval,
               rows_a, rows_b, gidx_a, gidx_b, ldst_a, ldst_b,
               zbuf, acc, sem_a, sem_b):
    wid = tile

    def zero_body(z, _):
        start = wid * ROWS_PT + z * ZROWS
        pltpu.sync_copy(zbuf, acc.at[pl.ds(start, ZROWS)])
        return 0
    lax.fori_loop(0, ROWS_PT // ZROWS, zero_body, 0)
    plsc.subcore_barrier()

    iota = _iota16()

    def stage_batch(k, gidx, ldst):
        def cp(j, _):
            gidx[pl.ds(j * LANES, LANES)] = cidx[pl.ds(k * BATCH + j * LANES, LANES)]
            ldst[pl.ds(j * LANES, LANES)] = cdst[pl.ds(k * BATCH + j * LANES, LANES)]
            return 0
        lax.fori_loop(0, BATCH // LANES, cp, 0)

    def start_gather(gidx, rows, sem):
        pltpu.async_copy(table_hbm.at[gidx], rows, sem)

    def wait_gather(gidx, rows, sem):
        pltpu.make_async_copy(table_hbm.at[gidx], rows, sem).wait()

    def scale_batch(k, rows):
        def sb(g, _):
            vvals = cval[pl.ds(k * BATCH + g * LANES, LANES)]
            for r16 in range(LANES):
                r = g * LANES + r16
                sv = vvals[r16]
                for q in range(H // LANES):
                    x = rows[r, pl.ds(q * LANES, LANES)]
                    rows[r, pl.ds(q * LANES, LANES)] = x * sv
            return 0
        lax.fori_loop(0, BATCH // LANES, sb, 0)

    def scatter_batch(rows, ldst):
        pltpu.sync_copy(rows, acc.at[ldst], add=True)

    def scan_chunk(sc_i, _):
        e0 = ebase + tile * EPT + sc_i * SCAN
        pltpu.sync_copy(dest_hbm.at[pl.ds(e0, SCAN)], dest_v)
        pltpu.sync_copy(src_hbm.at[pl.ds(e0, SCAN)], srcv_v)
        pltpu.sync_copy(val_hbm.at[pl.ds(e0, SCAN)], val_v)

        def scan_body(g, cnt):
            off = g * LANES
            d = dest_v[pl.ds(off, LANES)]
            m = (d >= base) & (d < base + CHUNK)
            plsc.store_compressed(cidx.at[pl.ds(cnt, LANES)],
                                  srcv_v[pl.ds(off, LANES)], mask=m)
            plsc.store_compressed(cdst.at[pl.ds(cnt, LANES)], d - base, mask=m)
            plsc.store_compressed(cval.at[pl.ds(cnt, LANES)],
                                  val_v[pl.ds(off, LANES)], mask=m)
            return cnt + jnp.sum(m.astype(jnp.int32))

        cnt = lax.fori_loop(0, GROUPS, scan_body, jnp.int32(0))

        for j in range(BATCH // LANES):
            pos = pl.ds(cnt + j * LANES, LANES)
            cidx[pos] = wid * BATCH + j * LANES + iota
            cdst[pos] = j * LANES + iota
            cval[pos] = jnp.zeros((LANES,), jnp.float32)

        nb = (cnt + (BATCH - 1)) // BATCH

        def batch_body(k, _):
            stage_batch(k, gidx_a, ldst_a)
            start_gather(gidx_a, rows_a, sem_a)
            wait_gather(gidx_a, rows_a, sem_a)
            scale_batch(k, rows_a)
            scatter_batch(rows_a, ldst_a)
            return 0

        lax.fori_loop(0, nb, batch_body, 0)
        return 0

    lax.fori_loop(0, NSCAN, scan_chunk, 0)
    plsc.subcore_barrier()

    def flush_body(z, _):
        start = wid * ROWS_PT + z * ZROWS
        pltpu.sync_copy(acc.at[pl.ds(start, ZROWS)],
                        out_hbm.at[pl.ds(base + start, ZROWS)])
        return 0
    lax.fori_loop(0, ROWS_PT // ZROWS, flush_body, 0)


def _sc_layer(user_tbl, item_tbl, edge_user, edge_item, edge_val):
    mesh = plsc.VectorSubcoreMesh(core_axis_name="c", subcore_axis_name="s",
                                  num_cores=NC, num_subcores=NS)

    @functools.partial(
        pl.kernel,
        out_type=(jax.ShapeDtypeStruct((NB, U_PAD, H), jnp.float32),
                  jax.ShapeDtypeStruct((NB, I_PAD, H), jnp.float32)),
        mesh=mesh,
        compiler_params=pltpu.CompilerParams(use_tc_tiling_on_sc=False,
                                             needs_layout_passes=False),
        scratch_types=[
            pltpu.VMEM((SCAN,), jnp.int32),
            pltpu.VMEM((SCAN,), jnp.int32),
            pltpu.VMEM((SCAN,), jnp.float32),
            pltpu.VMEM((CCAP,), jnp.int32),
            pltpu.VMEM((CCAP,), jnp.int32),
            pltpu.VMEM((CCAP,), jnp.float32),
            pltpu.VMEM((BATCH, H), jnp.float32),
            pltpu.VMEM((BATCH, H), jnp.float32),
            pltpu.VMEM((BATCH,), jnp.int32),
            pltpu.VMEM((BATCH,), jnp.int32),
            pltpu.VMEM((BATCH,), jnp.int32),
            pltpu.VMEM((BATCH,), jnp.int32),
            pltpu.VMEM((ZROWS, H), jnp.float32),
            pltpu.VMEM_SHARED((CHUNK, H), jnp.float32),
            pltpu.SemaphoreType.DMA,
            pltpu.SemaphoreType.DMA,
        ],
    )
    def k(user_hbm, item_hbm, eu_hbm, ei_hbm, ev_hbm, us_hbm, is_hbm,
          dest_v, srcv_v, val_v, cidx, cdst, cval,
          rows_a, rows_b, gidx_a, gidx_b, ldst_a, ldst_b,
          zbuf, acc, sem_a, sem_b):
        core = lax.axis_index("c")
        tile = lax.axis_index("s")

        def zfill_row(r, _):
            for q in range(H // LANES):
                zbuf[r, pl.ds(q * LANES, LANES)] = jnp.zeros((LANES,),
                                                             jnp.float32)
            return 0
        lax.fori_loop(0, ZROWS, zfill_row, 0)

        common = (dest_v, srcv_v, val_v, cidx, cdst, cval,
                  rows_a, rows_b, gidx_a, gidx_b, ldst_a, ldst_b,
                  zbuf, acc, sem_a, sem_b)

        for b in range(NB):
            ebase = b * E_PAD
            for p in range(U_PAD // (NC * CHUNK)):
                base = (2 * core + p) * CHUNK
                _emit_pass(eu_hbm, ei_hbm, ev_hbm,
                           item_hbm, us_hbm.at[b], base, ebase, tile, *common)
            base_i = core * CHUNK
            _emit_pass(ei_hbm, eu_hbm, ev_hbm,
                       user_hbm, is_hbm.at[b], base_i, ebase, tile, *common)

    return k(user_tbl, item_tbl, edge_user, edge_item, edge_val)


def _tc_layer1(stack, w):
    n = stack.shape[1]
    blk = 1024
    grid = (n // blk,)

    def body(stack_ref, w_ref, s_ref, e_ref):
        ws = w_ref[...]
        xs = [stack_ref[b] for b in range(NB)]
        for b in range(NB):
            s_ref[b] = _sigmoid(jnp.dot(xs[b], ws,
                                        preferred_element_type=jnp.float32))
        mean = (xs[0] + xs[1] + xs[2]) * (1.0 / 3.0)
        e_ref[...] = _sigmoid(jnp.dot(mean, ws,
                                      preferred_element_type=jnp.float32))

    return pl.pallas_call(
        body,
        grid=grid,
        in_specs=[
            pl.BlockSpec((NB, blk, H), lambda i: (0, i, 0)),
            pl.BlockSpec((H, H), lambda i: (0, 0)),
        ],
        out_specs=[
            pl.BlockSpec((NB, blk, H), lambda i: (0, i, 0)),
            pl.BlockSpec((blk, H), lambda i: (i, 0)),
        ],
        out_shape=[
            jax.ShapeDtypeStruct((NB, n, H), jnp.float32),
            jax.ShapeDtypeStruct((n, H), jnp.float32),
        ],
    )(stack, w)


def _tc_layer2_final(stack2, w2, e1, s1, w_cat_top, w_cat_bot):
    n = stack2.shape[1]
    blk = 1024
    grid = (n // blk,)

    def body(stack_ref, w2_ref, e1_ref, s1_ref, wt_ref, wb_ref,
             emb_ref, embs_ref):
        w2l = w2_ref[...]
        wt = wt_ref[...]
        wb = wb_ref[...]
        xs = [stack_ref[b] for b in range(NB)]
        for b in range(NB):
            s2 = _sigmoid(jnp.dot(xs[b], w2l,
                                  preferred_element_type=jnp.float32))
            embs_ref[b] = (jnp.dot(s1_ref[b], wt,
                                   preferred_element_type=jnp.float32)
                           + jnp.dot(s2, wb,
                                     preferred_element_type=jnp.float32))
        mean = (xs[0] + xs[1] + xs[2]) * (1.0 / 3.0)
        e2 = _sigmoid(jnp.dot(mean, w2l, preferred_element_type=jnp.float32))
        emb_ref[...] = (jnp.dot(e1_ref[...], wt,
                                preferred_element_type=jnp.float32)
                        + jnp.dot(e2, wb,
                                  preferred_element_type=jnp.float32))

    return pl.pallas_call(
        body,
        grid=grid,
        in_specs=[
            pl.BlockSpec((NB, blk, H), lambda i: (0, i, 0)),
            pl.BlockSpec((H, H), lambda i: (0, 0)),
            pl.BlockSpec((blk, H), lambda i: (i, 0)),
            pl.BlockSpec((NB, blk, H), lambda i: (0, i, 0)),
            pl.BlockSpec((H, H), lambda i: (0, 0)),
            pl.BlockSpec((H, H), lambda i: (0, 0)),
        ],
        out_specs=[
            pl.BlockSpec((blk, H), lambda i: (i, 0)),
            pl.BlockSpec((NB, blk, H), lambda i: (0, i, 0)),
        ],
        out_shape=[
            jax.ShapeDtypeStruct((n, H), jnp.float32),
            jax.ShapeDtypeStruct((NB, n, H), jnp.float32),
        ],
    )(stack2, w2, e1, s1, w_cat_top, w_cat_bot)


def kernel(user_table, item_table, u_w, i_w, u_cat_w, i_cat_w,
           edge_val, edge_user, edge_item):
    pad_n = E_PAD - E
    pad_ids = lax.iota(jnp.int32, pad_n)
    eu = jnp.concatenate(
        [edge_user.astype(jnp.int32),
         jnp.broadcast_to(pad_ids % U, (NB, pad_n))], axis=1).reshape(-1)
    ei = jnp.concatenate(
        [edge_item.astype(jnp.int32),
         jnp.broadcast_to(pad_ids % I, (NB, pad_n))], axis=1).reshape(-1)
    ev = jnp.concatenate(
        [edge_val, jnp.zeros((NB, pad_n), jnp.float32)], axis=1).reshape(-1)

    user_e = user_table
    item_e = item_table

    u_stack1, i_stack1 = _sc_layer(user_e, item_e, eu, ei, ev)
    u_s1, u_e1 = _tc_layer1(u_stack1, u_w[0])
    i_s1, i_e1 = _tc_layer1(i_stack1, i_w[0])

    u_stack2, i_stack2 = _sc_layer(u_e1, i_e1, eu, ei, ev)

    u_emb, u_embs = _tc_layer2_final(u_stack2, u_w[1], u_e1, u_s1,
                                     u_cat_w[:H], u_cat_w[H:])
    i_emb, i_embs = _tc_layer2_final(i_stack2, i_w[1], i_e1, i_s1,
                                     i_cat_w[:H], i_cat_w[H:])

    return (u_emb[:U], i_emb[:I], u_embs[:, :U, :], i_embs[:, :I, :])


# speedup vs baseline: 2.6388x; 1.2164x over previous
import functools

import jax
import jax.numpy as jnp
from jax import lax
from jax.experimental import pallas as pl
from jax.experimental.pallas import tpu as pltpu
from jax.experimental.pallas import tpu_sc as plsc

U = 100000
I = 50000
H = 64
NB = 3
E = 1000000
NL = 2

NC = 2
NS = 16
LANES = 16

E_PAD = 1048576
EPT = E_PAD // NS
SCAN = 2048
NSCAN = EPT // SCAN
GROUPS = SCAN // LANES

CHUNK = 25600
U_PAD = 4 * CHUNK
I_PAD = 2 * CHUNK
ROWS_PT = CHUNK // NS
ZROWS = 64

BATCH = 64
CCAP = SCAN + BATCH


def _sigmoid(x):
    return 1.0 / (1.0 + jnp.exp(-x))


def _iota16():
    return lax.broadcasted_iota(jnp.int32, (LANES,), 0)


def _emit_pass(dest_hbm, src_hbm, val_hbm, table_hbm, out_hbm, base, ebase,
               tile,
               dest_v, srcv_v, val_v, cidx, cdst, cval,
               rows_a, rows_b, rows_c, ldst_a, ldst_b, ldst_c,
               acc, sem_a, sem_b, sem_c):
    wid = tile

    def zfill_row(r, _):
        for q in range(H // LANES):
            rows_a[r, pl.ds(q * LANES, LANES)] = jnp.zeros((LANES,),
                                                           jnp.float32)
        return 0
    lax.fori_loop(0, ZROWS, zfill_row, 0)

    def zero_body(z, _):
        start = wid * ROWS_PT + z * ZROWS
        pltpu.sync_copy(rows_a, acc.at[pl.ds(start, ZROWS)])
        return 0
    lax.fori_loop(0, ROWS_PT // ZROWS, zero_body, 0)
    plsc.subcore_barrier()

    iota = _iota16()

    def stage_ldst(k, ldst):
        def cp(j, _):
            ldst[pl.ds(j * LANES, LANES)] = cdst[pl.ds(k * BATCH + j * LANES, LANES)]
            return 0
        lax.fori_loop(0, BATCH // LANES, cp, 0)

    def start_gather(k, rows, sem):
        pltpu.async_copy(table_hbm.at[cidx.at[pl.ds(k * BATCH, BATCH)]],
                         rows, sem)

    def wait_gather(k, rows, sem):
        pltpu.make_async_copy(table_hbm.at[cidx.at[pl.ds(k * BATCH, BATCH)]],
                              rows, sem).wait()

    def scale_batch(k, rows):
        def sb(g, _):
            vvals = cval[pl.ds(k * BATCH + g * LANES, LANES)]
            for r16 in range(LANES):
                r = g * LANES + r16
                sv = vvals[r16]
                for q in range(H // LANES):
                    x = rows[r, pl.ds(q * LANES, LANES)]
                    rows[r, pl.ds(q * LANES, LANES)] = x * sv
            return 0
        lax.fori_loop(0, BATCH // LANES, sb, 0)

    def scatter_batch(rows, ldst):
        pltpu.sync_copy(rows, acc.at[ldst], add=True)

    def scan_chunk(sc_i, _):
        e0 = ebase + tile * EPT + sc_i * SCAN
        pltpu.sync_copy(dest_hbm.at[pl.ds(e0, SCAN)], dest_v)
        pltpu.sync_copy(src_hbm.at[pl.ds(e0, SCAN)], srcv_v)
        pltpu.sync_copy(val_hbm.at[pl.ds(e0, SCAN)], val_v)

        def scan_body(g, cnt):
            off = g * LANES
            d = dest_v[pl.ds(off, LANES)]
            m = (d >= base) & (d < base + CHUNK)
            plsc.store_compressed(cidx.at[pl.ds(cnt, LANES)],
                                  srcv_v[pl.ds(off, LANES)], mask=m)
            plsc.store_compressed(cdst.at[pl.ds(cnt, LANES)], d - base, mask=m)
            plsc.store_compressed(cval.at[pl.ds(cnt, LANES)],
                                  val_v[pl.ds(off, LANES)], mask=m)
            return cnt + jnp.sum(m.astype(jnp.int32))

        cnt = lax.fori_loop(0, GROUPS, scan_body, jnp.int32(0))

        for j in range(BATCH // LANES):
            pos = pl.ds(cnt + j * LANES, LANES)
            cidx[pos] = wid * BATCH + j * LANES + iota
            cdst[pos] = j * LANES + iota
            cval[pos] = jnp.zeros((LANES,), jnp.float32)

        nb = (cnt + (BATCH - 1)) // BATCH

        def tri_body(t, _):
            k0 = 3 * t
            k1 = 3 * t + 1
            k2 = 3 * t + 2
            stage_ldst(k0, ldst_a)
            start_gather(k0, rows_a, sem_a)

            @pl.when(k1 < nb)
            def _():
                stage_ldst(k1, ldst_b)
                start_gather(k1, rows_b, sem_b)

            @pl.when(k2 < nb)
            def _():
                stage_ldst(k2, ldst_c)
                start_gather(k2, rows_c, sem_c)

            wait_gather(k0, rows_a, sem_a)
            scale_batch(k0, rows_a)
            scatter_batch(rows_a, ldst_a)

            @pl.when(k1 < nb)
            def _():
                wait_gather(k1, rows_b, sem_b)
                scale_batch(k1, rows_b)
                scatter_batch(rows_b, ldst_b)

            @pl.when(k2 < nb)
            def _():
                wait_gather(k2, rows_c, sem_c)
                scale_batch(k2, rows_c)
                scatter_batch(rows_c, ldst_c)
            return 0

        lax.fori_loop(0, (nb + 2) // 3, tri_body, 0)
        return 0

    lax.fori_loop(0, NSCAN, scan_chunk, 0)
    plsc.subcore_barrier()

    def flush_body(z, _):
        start = wid * ROWS_PT + z * ZROWS
        pltpu.sync_copy(acc.at[pl.ds(start, ZROWS)],
                        out_hbm.at[pl.ds(base + start, ZROWS)])
        return 0
    lax.fori_loop(0, ROWS_PT // ZROWS, flush_body, 0)


def _sc_layer(user_tbl, item_tbl, edge_user, edge_item, edge_val):
    mesh = plsc.VectorSubcoreMesh(core_axis_name="c", subcore_axis_name="s",
                                  num_cores=NC, num_subcores=NS)

    @functools.partial(
        pl.kernel,
        out_type=(jax.ShapeDtypeStruct((NB, U_PAD, H), jnp.float32),
                  jax.ShapeDtypeStruct((NB, I_PAD, H), jnp.float32)),
        mesh=mesh,
        compiler_params=pltpu.CompilerParams(use_tc_tiling_on_sc=False,
                                             needs_layout_passes=False),
        scratch_types=[
            pltpu.VMEM((SCAN,), jnp.int32),
            pltpu.VMEM((SCAN,), jnp.int32),
            pltpu.VMEM((SCAN,), jnp.float32),
            pltpu.VMEM((CCAP,), jnp.int32),
            pltpu.VMEM((CCAP,), jnp.int32),
            pltpu.VMEM((CCAP,), jnp.float32),
            pltpu.VMEM((BATCH, H), jnp.float32),
            pltpu.VMEM((BATCH, H), jnp.float32),
            pltpu.VMEM((BATCH, H), jnp.float32),
            pltpu.VMEM((BATCH,), jnp.int32),
            pltpu.VMEM((BATCH,), jnp.int32),
            pltpu.VMEM((BATCH,), jnp.int32),
            pltpu.VMEM_SHARED((CHUNK, H), jnp.float32),
            pltpu.SemaphoreType.DMA,
            pltpu.SemaphoreType.DMA,
            pltpu.SemaphoreType.DMA,
        ],
    )
    def k(user_hbm, item_hbm, eu_hbm, ei_hbm, ev_hbm, us_hbm, is_hbm,
          dest_v, srcv_v, val_v, cidx, cdst, cval,
          rows_a, rows_b, rows_c, ldst_a, ldst_b, ldst_c,
          acc, sem_a, sem_b, sem_c):
        core = lax.axis_index("c")
        tile = lax.axis_index("s")

        common = (dest_v, srcv_v, val_v, cidx, cdst, cval,
                  rows_a, rows_b, rows_c, ldst_a, ldst_b, ldst_c,
                  acc, sem_a, sem_b, sem_c)

        for b in range(NB):
            ebase = b * E_PAD
            for p in range(U_PAD // (NC * CHUNK)):
                base = (2 * core + p) * CHUNK
                _emit_pass(eu_hbm, ei_hbm, ev_hbm,
                           item_hbm, us_hbm.at[b], base, ebase, tile, *common)
            base_i = core * CHUNK
            _emit_pass(ei_hbm, eu_hbm, ev_hbm,
                       user_hbm, is_hbm.at[b], base_i, ebase, tile, *common)

    return k(user_tbl, item_tbl, edge_user, edge_item, edge_val)


def _tc_layer1(stack, w):
    n = stack.shape[1]
    blk = 1024
    grid = (n // blk,)

    def body(stack_ref, w_ref, s_ref, e_ref):
        ws = w_ref[...]
        xs = [stack_ref[b] for b in range(NB)]
        for b in range(NB):
            s_ref[b] = _sigmoid(jnp.dot(xs[b], ws,
                                        preferred_element_type=jnp.float32))
        mean = (xs[0] + xs[1] + xs[2]) * (1.0 / 3.0)
        e_ref[...] = _sigmoid(jnp.dot(mean, ws,
                                      preferred_element_type=jnp.float32))

    return pl.pallas_call(
        body,
        grid=grid,
        in_specs=[
            pl.BlockSpec((NB, blk, H), lambda i: (0, i, 0)),
            pl.BlockSpec((H, H), lambda i: (0, 0)),
        ],
        out_specs=[
            pl.BlockSpec((NB, blk, H), lambda i: (0, i, 0)),
            pl.BlockSpec((blk, H), lambda i: (i, 0)),
        ],
        out_shape=[
            jax.ShapeDtypeStruct((NB, n, H), jnp.float32),
            jax.ShapeDtypeStruct((n, H), jnp.float32),
        ],
    )(stack, w)


def _tc_layer2_final(stack2, w2, e1, s1, w_cat_top, w_cat_bot):
    n = stack2.shape[1]
    blk = 1024
    grid = (n // blk,)

    def body(stack_ref, w2_ref, e1_ref, s1_ref, wt_ref, wb_ref,
             emb_ref, embs_ref):
        w2l = w2_ref[...]
        wt = wt_ref[...]
        wb = wb_ref[...]
        xs = [stack_ref[b] for b in range(NB)]
        for b in range(NB):
            s2 = _sigmoid(jnp.dot(xs[b], w2l,
                                  preferred_element_type=jnp.float32))
            embs_ref[b] = (jnp.dot(s1_ref[b], wt,
                                   preferred_element_type=jnp.float32)
                           + jnp.dot(s2, wb,
                                     preferred_element_type=jnp.float32))
        mean = (xs[0] + xs[1] + xs[2]) * (1.0 / 3.0)
        e2 = _sigmoid(jnp.dot(mean, w2l, preferred_element_type=jnp.float32))
        emb_ref[...] = (jnp.dot(e1_ref[...], wt,
                                preferred_element_type=jnp.float32)
                        + jnp.dot(e2, wb,
                                  preferred_element_type=jnp.float32))

    return pl.pallas_call(
        body,
        grid=grid,
        in_specs=[
            pl.BlockSpec((NB, blk, H), lambda i: (0, i, 0)),
            pl.BlockSpec((H, H), lambda i: (0, 0)),
            pl.BlockSpec((blk, H), lambda i: (i, 0)),
            pl.BlockSpec((NB, blk, H), lambda i: (0, i, 0)),
            pl.BlockSpec((H, H), lambda i: (0, 0)),
            pl.BlockSpec((H, H), lambda i: (0, 0)),
        ],
        out_specs=[
            pl.BlockSpec((blk, H), lambda i: (i, 0)),
            pl.BlockSpec((NB, blk, H), lambda i: (0, i, 0)),
        ],
        out_shape=[
            jax.ShapeDtypeStruct((n, H), jnp.float32),
            jax.ShapeDtypeStruct((NB, n, H), jnp.float32),
        ],
    )(stack2, w2, e1, s1, w_cat_top, w_cat_bot)


def kernel(user_table, item_table, u_w, i_w, u_cat_w, i_cat_w,
           edge_val, edge_user, edge_item):
    pad_n = E_PAD - E
    pad_ids = lax.iota(jnp.int32, pad_n)
    eu = jnp.concatenate(
        [edge_user.astype(jnp.int32),
         jnp.broadcast_to(pad_ids % U, (NB, pad_n))], axis=1).reshape(-1)
    ei = jnp.concatenate(
        [edge_item.astype(jnp.int32),
         jnp.broadcast_to(pad_ids % I, (NB, pad_n))], axis=1).reshape(-1)
    ev = jnp.concatenate(
        [edge_val, jnp.zeros((NB, pad_n), jnp.float32)], axis=1).reshape(-1)

    user_e = user_table
    item_e = item_table

    u_stack1, i_stack1 = _sc_layer(user_e, item_e, eu, ei, ev)
    u_s1, u_e1 = _tc_layer1(u_stack1, u_w[0])
    i_s1, i_e1 = _tc_layer1(i_stack1, i_w[0])

    u_stack2, i_stack2 = _sc_layer(u_e1, i_e1, eu, ei, ev)

    u_emb, u_embs = _tc_layer2_final(u_stack2, u_w[1], u_e1, u_s1,
                                     u_cat_w[:H], u_cat_w[H:])
    i_emb, i_embs = _tc_layer2_final(i_stack2, i_w[1], i_e1, i_s1,
                                     i_cat_w[:H], i_cat_w[H:])

    return (u_emb[:U], i_emb[:I], u_embs[:, :U, :], i_embs[:, :I, :])


# speedup vs baseline: 4.8205x; 1.8268x over previous
import functools

import jax
import jax.numpy as jnp
from jax import lax
from jax.experimental import pallas as pl
from jax.experimental.pallas import tpu as pltpu
from jax.experimental.pallas import tpu_sc as plsc

U = 100000
I = 50000
H = 64
NB = 3
E = 1000000
NL = 2

NC = 2
NS = 16
LANES = 16

E_PAD = 1048576
EPT = E_PAD // NS
SCAN = 1024
NSCAN = EPT // SCAN
GROUPS = SCAN // LANES

CHUNK = 25600
U_PAD = 4 * CHUNK
I_PAD = 2 * CHUNK
ROWS_PT = CHUNK // NS

BATCH = 128
CCAP = SCAN + 2 * BATCH


def _sigmoid(x):
    return 1.0 / (1.0 + jnp.exp(-x))


def _iota16():
    return lax.broadcasted_iota(jnp.int32, (LANES,), 0)


def _emit_pass(dest_hbm, src_hbm, val_hbm, table_hbm, out_hbm, base, ebase,
               tile,
               dest_v, srcv_v, val_v, cidx, cdst, cval,
               rows_a, rows_b, ldst_a, ldst_b,
               acc, sem_ga, sem_gb, sem_sa, sem_sb, sem_st):
    wid = tile

    def zfill_row(r, _):
        for q in range(H // LANES):
            rows_a[r, pl.ds(q * LANES, LANES)] = jnp.zeros((LANES,),
                                                           jnp.float32)
        return 0
    lax.fori_loop(0, BATCH, zfill_row, 0)

    def zero_body(z, _):
        start = wid * ROWS_PT + z * BATCH
        pltpu.sync_copy(rows_a, acc.at[pl.ds(start, BATCH)])
        return 0
    lax.fori_loop(0, ROWS_PT // BATCH, zero_body, 0)
    pltpu.sync_copy(rows_a.at[pl.ds(0, ROWS_PT % BATCH)],
                    acc.at[pl.ds(wid * ROWS_PT + (ROWS_PT // BATCH) * BATCH,
                                 ROWS_PT % BATCH)])
    plsc.subcore_barrier()

    iota = _iota16()

    def stage_ldst(k, ldst):
        def cp(j, _):
            ldst[pl.ds(j * LANES, LANES)] = cdst[pl.ds(k * BATCH + j * LANES,
                                                       LANES)]
            return 0
        lax.fori_loop(0, BATCH // LANES, cp, 0)

    def start_gather(k, rows, sem):
        pltpu.async_copy(table_hbm.at[cidx.at[pl.ds(k * BATCH, BATCH)]],
                         rows, sem)

    def wait_gather(k, rows, sem):
        pltpu.make_async_copy(table_hbm.at[cidx.at[pl.ds(k * BATCH, BATCH)]],
                              rows, sem).wait()

    def scale_batch(k, rows):
        def sb(g, _):
            vvals = cval[pl.ds(k * BATCH + g * LANES, LANES)]
            for r16 in range(LANES):
                r = g * LANES + r16
                sv = vvals[r16]
                for q in range(H // LANES):
                    x = rows[r, pl.ds(q * LANES, LANES)]
                    rows[r, pl.ds(q * LANES, LANES)] = x * sv
            return 0
        lax.fori_loop(0, BATCH // LANES, sb, 0)

    def start_scatter(rows, ldst, sem):
        pltpu.async_copy(rows, acc.at[ldst], sem, add=True)

    def wait_scatter(rows, ldst, sem):
        pltpu.make_async_copy(rows, acc.at[ldst], sem).wait()

    def drain(nb):
        def pair_body(p, _):
            k0 = 2 * p
            k1 = 2 * p + 1
            stage_ldst(k0, ldst_a)
            start_gather(k0, rows_a, sem_ga)

            @pl.when(k1 < nb)
            def _():
                stage_ldst(k1, ldst_b)
                start_gather(k1, rows_b, sem_gb)

            wait_gather(k0, rows_a, sem_ga)
            scale_batch(k0, rows_a)
            start_scatter(rows_a, ldst_a, sem_sa)

            @pl.when(k1 < nb)
            def _():
                wait_gather(k1, rows_b, sem_gb)
                scale_batch(k1, rows_b)
                start_scatter(rows_b, ldst_b, sem_sb)

            wait_scatter(rows_a, ldst_a, sem_sa)

            @pl.when(k1 < nb)
            def _():
                wait_scatter(rows_b, ldst_b, sem_sb)
            return 0

        lax.fori_loop(0, (nb + 1) // 2, pair_body, 0)

    def scan_chunk(sc_i, cnt_in):
        e0 = ebase + tile * EPT + sc_i * SCAN
        pltpu.async_copy(dest_hbm.at[pl.ds(e0, SCAN)], dest_v, sem_st)
        pltpu.async_copy(src_hbm.at[pl.ds(e0, SCAN)], srcv_v, sem_st)
        pltpu.async_copy(val_hbm.at[pl.ds(e0, SCAN)], val_v, sem_st)
        pltpu.make_async_copy(dest_hbm.at[pl.ds(e0, SCAN)], dest_v,
                              sem_st).wait()
        pltpu.make_async_copy(src_hbm.at[pl.ds(e0, SCAN)], srcv_v,
                              sem_st).wait()
        pltpu.make_async_copy(val_hbm.at[pl.ds(e0, SCAN)], val_v,
                              sem_st).wait()

        def scan_body(g, cnt):
            off = g * LANES
            d = dest_v[pl.ds(off, LANES)]
            m = (d >= base) & (d < base + CHUNK)
            plsc.store_compressed(cidx.at[pl.ds(cnt, LANES)],
                                  srcv_v[pl.ds(off, LANES)], mask=m)
            plsc.store_compressed(cdst.at[pl.ds(cnt, LANES)], d - base,
                                  mask=m)
            plsc.store_compressed(cval.at[pl.ds(cnt, LANES)],
                                  val_v[pl.ds(off, LANES)], mask=m)
            return cnt + jnp.sum(m.astype(jnp.int32))

        cnt = lax.fori_loop(0, GROUPS, scan_body, cnt_in)

        nb = cnt // BATCH
        drain(nb)

        rem = cnt - nb * BATCH

        @pl.when(nb > 0)
        def _():
            def mv(j, _):
                pos_src = pl.ds(nb * BATCH + j * LANES, LANES)
                pos_dst = pl.ds(j * LANES, LANES)
                cidx[pos_dst] = cidx[pos_src]
                cdst[pos_dst] = cdst[pos_src]
                cval[pos_dst] = cval[pos_src]
                return 0
            lax.fori_loop(0, BATCH // LANES, mv, 0)

        return rem

    rem = lax.fori_loop(0, NSCAN, scan_chunk, jnp.int32(0))

    @pl.when(rem > 0)
    def _():
        for j in range(BATCH // LANES):
            pos = pl.ds(rem + j * LANES, LANES)
            cidx[pos] = wid * BATCH + j * LANES + iota
            cdst[pos] = j * LANES + iota
            cval[pos] = jnp.zeros((LANES,), jnp.float32)
        drain(1)

    plsc.subcore_barrier()

    def flush_body(z, _):
        start = wid * ROWS_PT + z * BATCH
        pltpu.sync_copy(acc.at[pl.ds(start, BATCH)],
                        out_hbm.at[pl.ds(base + start, BATCH)])
        return 0
    lax.fori_loop(0, ROWS_PT // BATCH, flush_body, 0)
    tail = ROWS_PT % BATCH
    tstart = wid * ROWS_PT + (ROWS_PT // BATCH) * BATCH
    pltpu.sync_copy(acc.at[pl.ds(tstart, tail)],
                    out_hbm.at[pl.ds(base + tstart, tail)])


def _sc_layer(user_tbl, item_tbl, edge_user, edge_item, edge_val):
    mesh = plsc.VectorSubcoreMesh(core_axis_name="c", subcore_axis_name="s",
                                  num_cores=NC, num_subcores=NS)

    @functools.partial(
        pl.kernel,
        out_type=(jax.ShapeDtypeStruct((NB, U_PAD, H), jnp.float32),
                  jax.ShapeDtypeStruct((NB, I_PAD, H), jnp.float32)),
        mesh=mesh,
        compiler_params=pltpu.CompilerParams(use_tc_tiling_on_sc=False,
                                             needs_layout_passes=False),
        scratch_types=[
            pltpu.VMEM((SCAN,), jnp.int32),
            pltpu.VMEM((SCAN,), jnp.int32),
            pltpu.VMEM((SCAN,), jnp.float32),
            pltpu.VMEM((CCAP,), jnp.int32),
            pltpu.VMEM((CCAP,), jnp.int32),
            pltpu.VMEM((CCAP,), jnp.float32),
            pltpu.VMEM((BATCH, H), jnp.float32),
            pltpu.VMEM((BATCH, H), jnp.float32),
            pltpu.VMEM((BATCH,), jnp.int32),
            pltpu.VMEM((BATCH,), jnp.int32),
            pltpu.VMEM_SHARED((CHUNK, H), jnp.float32),
            pltpu.SemaphoreType.DMA,
            pltpu.SemaphoreType.DMA,
            pltpu.SemaphoreType.DMA,
            pltpu.SemaphoreType.DMA,
            pltpu.SemaphoreType.DMA,
        ],
    )
    def k(user_hbm, item_hbm, eu_hbm, ei_hbm, ev_hbm, us_hbm, is_hbm,
          dest_v, srcv_v, val_v, cidx, cdst, cval,
          rows_a, rows_b, ldst_a, ldst_b,
          acc, sem_ga, sem_gb, sem_sa, sem_sb, sem_st):
        core = lax.axis_index("c")
        tile = lax.axis_index("s")

        common = (dest_v, srcv_v, val_v, cidx, cdst, cval,
                  rows_a, rows_b, ldst_a, ldst_b,
                  acc, sem_ga, sem_gb, sem_sa, sem_sb, sem_st)

        for b in range(NB):
            ebase = b * E_PAD
            for p in range(U_PAD // (NC * CHUNK)):
                base = (2 * core + p) * CHUNK
                _emit_pass(eu_hbm, ei_hbm, ev_hbm,
                           item_hbm, us_hbm.at[b], base, ebase, tile, *common)
            base_i = core * CHUNK
            _emit_pass(ei_hbm, eu_hbm, ev_hbm,
                       user_hbm, is_hbm.at[b], base_i, ebase, tile, *common)

    return k(user_tbl, item_tbl, edge_user, edge_item, edge_val)


def _tc_layer1(stack, w):
    n = stack.shape[1]
    blk = 1024
    grid = (n // blk,)

    def body(stack_ref, w_ref, s_ref, e_ref):
        ws = w_ref[...]
        xs = [stack_ref[b] for b in range(NB)]
        for b in range(NB):
            s_ref[b] = _sigmoid(jnp.dot(xs[b], ws,
                                        preferred_element_type=jnp.float32))
        mean = (xs[0] + xs[1] + xs[2]) * (1.0 / 3.0)
        e_ref[...] = _sigmoid(jnp.dot(mean, ws,
                                      preferred_element_type=jnp.float32))

    return pl.pallas_call(
        body,
        grid=grid,
        in_specs=[
            pl.BlockSpec((NB, blk, H), lambda i: (0, i, 0)),
            pl.BlockSpec((H, H), lambda i: (0, 0)),
        ],
        out_specs=[
            pl.BlockSpec((NB, blk, H), lambda i: (0, i, 0)),
            pl.BlockSpec((blk, H), lambda i: (i, 0)),
        ],
        out_shape=[
            jax.ShapeDtypeStruct((NB, n, H), jnp.float32),
            jax.ShapeDtypeStruct((n, H), jnp.float32),
        ],
    )(stack, w)


def _tc_layer2_final(stack2, w2, e1, s1, w_cat_top, w_cat_bot):
    n = stack2.shape[1]
    blk = 1024
    grid = (n // blk,)

    def body(stack_ref, w2_ref, e1_ref, s1_ref, wt_ref, wb_ref,
             emb_ref, embs_ref):
        w2l = w2_ref[...]
        wt = wt_ref[...]
        wb = wb_ref[...]
        xs = [stack_ref[b] for b in range(NB)]
        for b in range(NB):
            s2 = _sigmoid(jnp.dot(xs[b], w2l,
                                  preferred_element_type=jnp.float32))
            embs_ref[b] = (jnp.dot(s1_ref[b], wt,
                                   preferred_element_type=jnp.float32)
                           + jnp.dot(s2, wb,
                                     preferred_element_type=jnp.float32))
        mean = (xs[0] + xs[1] + xs[2]) * (1.0 / 3.0)
        e2 = _sigmoid(jnp.dot(mean, w2l, preferred_element_type=jnp.float32))
        emb_ref[...] = (jnp.dot(e1_ref[...], wt,
                                preferred_element_type=jnp.float32)
                        + jnp.dot(e2, wb,
                                  preferred_element_type=jnp.float32))

    return pl.pallas_call(
        body,
        grid=grid,
        in_specs=[
            pl.BlockSpec((NB, blk, H), lambda i: (0, i, 0)),
            pl.BlockSpec((H, H), lambda i: (0, 0)),
            pl.BlockSpec((blk, H), lambda i: (i, 0)),
            pl.BlockSpec((NB, blk, H), lambda i: (0, i, 0)),
            pl.BlockSpec((H, H), lambda i: (0, 0)),
            pl.BlockSpec((H, H), lambda i: (0, 0)),
        ],
        out_specs=[
            pl.BlockSpec((blk, H), lambda i: (i, 0)),
            pl.BlockSpec((NB, blk, H), lambda i: (0, i, 0)),
        ],
        out_shape=[
            jax.ShapeDtypeStruct((n, H), jnp.float32),
            jax.ShapeDtypeStruct((NB, n, H), jnp.float32),
        ],
    )(stack2, w2, e1, s1, w_cat_top, w_cat_bot)


def kernel(user_table, item_table, u_w, i_w, u_cat_w, i_cat_w,
           edge_val, edge_user, edge_item):
    pad_n = E_PAD - E
    pad_ids = lax.iota(jnp.int32, pad_n)
    eu = jnp.concatenate(
        [edge_user.astype(jnp.int32),
         jnp.broadcast_to(pad_ids % U, (NB, pad_n))], axis=1).reshape(-1)
    ei = jnp.concatenate(
        [edge_item.astype(jnp.int32),
         jnp.broadcast_to(pad_ids % I, (NB, pad_n))], axis=1).reshape(-1)
    ev = jnp.concatenate(
        [edge_val, jnp.zeros((NB, pad_n), jnp.float32)], axis=1).reshape(-1)

    user_e = user_table
    item_e = item_table

    u_stack1, i_stack1 = _sc_layer(user_e, item_e, eu, ei, ev)
    u_s1, u_e1 = _tc_layer1(u_stack1, u_w[0])
    i_s1, i_e1 = _tc_layer1(i_stack1, i_w[0])

    u_stack2, i_stack2 = _sc_layer(u_e1, i_e1, eu, ei, ev)

    u_emb, u_embs = _tc_layer2_final(u_stack2, u_w[1], u_e1, u_s1,
                                     u_cat_w[:H], u_cat_w[H:])
    i_emb, i_embs = _tc_layer2_final(i_stack2, i_w[1], i_e1, i_s1,
                                     i_cat_w[:H], i_cat_w[H:])

    return (u_emb[:U], i_emb[:I], u_embs[:, :U, :], i_embs[:, :I, :])
